# baseline (speedup 1.0000x reference)
import functools

import jax
import jax.numpy as jnp
from jax import lax
from jax.experimental import pallas as pl
from jax.experimental.pallas import tpu as pltpu
from jax.experimental.pallas import tpu_sc as plsc

D = 128
NC, NS = 2, 16
NW = NC * NS
N_A, N_M = 1000, 10000
NUM_MODS, NUM_PREDS = 6, 30


SC_K = 128


def _pad_edges(idx, fill):
    e = idx.shape[0]
    ep = -(-e // (NW * SC_K)) * (NW * SC_K)
    if ep != e:
        idx = jnp.concatenate([idx, jnp.full((ep - e,), fill, idx.dtype)])
    return idx


def _seg_pad(nseg):
    rpt = -(-nseg // NS)
    rpt = (rpt + 7) // 8 * 8
    return rpt, rpt * NS


def _blk(m):
    for b in (512, 256, 200, 128, 8):
        if m % b == 0:
            return b
    raise ValueError(m)


def _enc_body(x, w, o):
    o[...] = jax.nn.relu(x[...] @ w[...])


def _tc_encode(x, W):
    m = x.shape[0]
    b = _blk(m)
    return pl.pallas_call(
        _enc_body,
        grid=(m // b,),
        in_specs=[pl.BlockSpec((b, D), lambda i: (i, 0)),
                  pl.BlockSpec((D, D), lambda i: (0, 0))],
        out_specs=pl.BlockSpec((b, D), lambda i: (i, 0)),
        out_shape=jax.ShapeDtypeStruct((m, D), jnp.float32),
    )(x, W)


def _mm_body(x, w, o):
    o[...] = x[...] @ w[...]


def _tc_matmul(x, W):
    m = x.shape[0]
    b = _blk(m)
    return pl.pallas_call(
        _mm_body,
        grid=(m // b,),
        in_specs=[pl.BlockSpec((b, D), lambda i: (i, 0)),
                  pl.BlockSpec((D, D), lambda i: (0, 0))],
        out_specs=pl.BlockSpec((b, D), lambda i: (i, 0)),
        out_shape=jax.ShapeDtypeStruct((m, D), jnp.float32),
    )(x, W)


def _map_upd_body(x, w1, a0, a1, o):
    o[...] = jax.nn.relu(x[...] @ w1[...] + (a0[...] + a1[...])) + x[...]


def _tc_map_update(x, W1, p0, p1):
    m = x.shape[0]
    b = _blk(m)
    return pl.pallas_call(
        _map_upd_body,
        grid=(m // b,),
        in_specs=[pl.BlockSpec((b, D), lambda i: (i, 0)),
                  pl.BlockSpec((D, D), lambda i: (0, 0)),
                  pl.BlockSpec((b, D), lambda i: (i, 0)),
                  pl.BlockSpec((b, D), lambda i: (i, 0))],
        out_specs=pl.BlockSpec((b, D), lambda i: (i, 0)),
        out_shape=jax.ShapeDtypeStruct((m, D), jnp.float32),
    )(x, W1, p0, p1)


def _att_upd_body(x, wq, wo, a0, a1, o):
    o[...] = jax.nn.relu(x[...] @ wq[...] + (a0[...] + a1[...]) @ wo[...]) + x[...]


def _tc_att_update(x, Wq, Wo, p0, p1):
    m = x.shape[0]
    b = _blk(m)
    return pl.pallas_call(
        _att_upd_body,
        grid=(m // b,),
        in_specs=[pl.BlockSpec((b, D), lambda i: (i, 0)),
                  pl.BlockSpec((D, D), lambda i: (0, 0)),
                  pl.BlockSpec((D, D), lambda i: (0, 0)),
                  pl.BlockSpec((b, D), lambda i: (i, 0)),
                  pl.BlockSpec((b, D), lambda i: (i, 0))],
        out_specs=pl.BlockSpec((b, D), lambda i: (i, 0)),
        out_shape=jax.ShapeDtypeStruct((m, D), jnp.float32),
    )(x, Wq, Wo, p0, p1)


def _proj_cs_body(x, cp, wt, wd, o):
    o[:, :D] = x[...] @ wt[...]
    o[:, D:] = cp[...] @ wd[...]


def _tc_proj_cs(x, cp, Wt, Wd):
    m = x.shape[0]
    b = _blk(m)
    return pl.pallas_call(
        _proj_cs_body,
        grid=(m // b,),
        in_specs=[pl.BlockSpec((b, D), lambda i: (i, 0)),
                  pl.BlockSpec((b, D), lambda i: (i, 0)),
                  pl.BlockSpec((D, D), lambda i: (0, 0)),
                  pl.BlockSpec((D, D), lambda i: (0, 0))],
        out_specs=pl.BlockSpec((b, 2 * D), lambda i: (i, 0)),
        out_shape=jax.ShapeDtypeStruct((m, 2 * D), jnp.float32),
    )(x, cp, Wt, Wd)


def _msg_body(gcs, ad, wcb, o):
    df = jax.nn.relu(ad[...] - gcs[:, D:])
    o[...] = jax.nn.relu(gcs[:, :D] + df @ wcb[...])


def _tc_msg(GCS, AD, Wcb):
    e = GCS.shape[0]
    b = 256
    return pl.pallas_call(
        _msg_body,
        grid=(e // b,),
        in_specs=[pl.BlockSpec((b, 2 * D), lambda i: (i, 0)),
                  pl.BlockSpec((b, D), lambda i: (i, 0)),
                  pl.BlockSpec((D, D), lambda i: (0, 0))],
        out_specs=pl.BlockSpec((b, D), lambda i: (i, 0)),
        out_shape=jax.ShapeDtypeStruct((e, D), jnp.float32),
    )(GCS, AD, Wcb)


def _pred_body(x, w, bvec, o):
    o[...] = x[...] @ w[...] + bvec[...]


def _tc_pred(x, W, bvec):
    m, f = x.shape[0], W.shape[1]
    b = _blk(m)
    return pl.pallas_call(
        _pred_body,
        grid=(m // b,),
        in_specs=[pl.BlockSpec((b, D), lambda i: (i, 0)),
                  pl.BlockSpec((D, f), lambda i: (0, 0)),
                  pl.BlockSpec((1, f), lambda i: (0, 0))],
        out_specs=pl.BlockSpec((b, f), lambda i: (i, 0)),
        out_shape=jax.ShapeDtypeStruct((m, f), jnp.float32),
    )(x, W, bvec)


@functools.lru_cache(maxsize=1)
def _sc_mesh():
    return plsc.VectorSubcoreMesh(core_axis_name="c", subcore_axis_name="s",
                                  num_cores=NC, num_subcores=NS)


SC_NBUF = 4
SC_CK = 64


def _sc_gather2(gcs_t, ad_t, src, dst):
    e = src.shape[0]
    ew = e // NW
    k = SC_CK
    nbuf = SC_NBUF
    nch = ew // k
    assert nch >= nbuf and nch * k == ew
    src2 = src.reshape(NW * nch, k)
    dst2 = dst.reshape(NW * nch, k)

    scratch = ([pltpu.VMEM((k,), jnp.int32)] * (2 * nbuf)
               + [pltpu.VMEM((k, 2 * D), jnp.float32)] * nbuf
               + [pltpu.VMEM((k, D), jnp.float32)] * nbuf
               + [pltpu.SemaphoreType.DMA((nbuf,))] * 3)

    @functools.partial(
        pl.kernel,
        out_type=(jax.ShapeDtypeStruct((e, 2 * D), jnp.float32),
                  jax.ShapeDtypeStruct((e, D), jnp.float32)),
        mesh=_sc_mesh(),
        scratch_types=scratch,
    )
    def body(gcst_h, adt_h, src_h, dst_h, gcs_h, ad_h, *sc):
        isrc = sc[0:nbuf]
        idst = sc[nbuf:2 * nbuf]
        gb = sc[2 * nbuf:3 * nbuf]
        ab = sc[3 * nbuf:4 * nbuf]
        isem, gsem, osem = sc[4 * nbuf:4 * nbuf + 3]
        c = lax.axis_index("c")
        s = lax.axis_index("s")
        w = c * NS + s
        base = w * ew

        def idx_load(i, h):
            pltpu.async_copy(src_h.at[w * nch + i], isrc[h], isem.at[h])
            pltpu.async_copy(dst_h.at[w * nch + i], idst[h], isem.at[h])

        def idx_wait(h):
            pltpu.make_async_copy(src_h.at[0], isrc[h], isem.at[h]).wait()
            pltpu.make_async_copy(dst_h.at[0], idst[h], isem.at[h]).wait()

        def g_fire(h):
            pltpu.async_copy(gcst_h.at[isrc[h]], gb[h], gsem.at[h])
            pltpu.async_copy(adt_h.at[idst[h]], ab[h], gsem.at[h])

        def g_wait(h):
            pltpu.make_async_copy(gcst_h.at[isrc[h]], gb[h], gsem.at[h]).wait()
            pltpu.make_async_copy(adt_h.at[idst[h]], ab[h], gsem.at[h]).wait()

        def out_fire(i, h):
            off = base + i * k
            pltpu.async_copy(gb[h], gcs_h.at[pl.ds(off, k)], osem.at[h])
            pltpu.async_copy(ab[h], ad_h.at[pl.ds(off, k)], osem.at[h])

        def out_wait(h):
            pltpu.make_async_copy(gb[h], gcs_h.at[pl.ds(0, k)],
                                  osem.at[h]).wait()
            pltpu.make_async_copy(ab[h], ad_h.at[pl.ds(0, k)],
                                  osem.at[h]).wait()

        def chunk_body(x, sh, traced):
            nxt = x + nbuf - 1
            so = (sh - 1) % nbuf

            def fire_next():
                def reuse():
                    out_wait(so)
                if traced:
                    pl.when(nxt >= nbuf)(reuse)
                elif nxt >= nbuf:
                    reuse()
                idx_wait(so)
                g_fire(so)

            if traced:
                pl.when(nxt < nch)(fire_next)
            elif nxt < nch:
                fire_next()
            g_wait(sh)
            out_fire(x, sh)

            def load_next():
                idx_load(x + nbuf, sh)
            if traced:
                pl.when(x + nbuf < nch)(load_next)
            elif x + nbuf < nch:
                load_next()

        for j in range(nbuf):
            idx_load(j, j)
        for j in range(nbuf - 1):
            idx_wait(j)
            g_fire(j)

        def block(g, _):
            xb = g * nbuf
            for sh in range(nbuf):
                chunk_body(xb + sh, sh, True)
            return 0

        nmain = (nch // nbuf) * nbuf
        lax.fori_loop(0, nch // nbuf, block, 0)
        for j in range(nch % nbuf):
            chunk_body(nmain + j, (nmain + j) % nbuf, False)
        for j in range(nbuf):
            out_wait((nch - nbuf + j) % nbuf)

    return body(gcs_t, ad_t, src2, dst2)


def _sc_segsum(rows_arr, dst, nseg, table=None, src=None):
    e = dst.shape[0]
    ew = e // NW
    k = SC_CK
    nbuf = SC_NBUF
    nch = ew // k
    assert nch >= nbuf and nch * k == ew
    gather = table is not None
    rpt, npad = _seg_pad(nseg)
    zeros = jnp.zeros((rpt, D), jnp.float32)
    dst2 = dst.reshape(NW * nch, k)
    ins = (table, src.reshape(NW * nch, k), dst2, zeros) if gather \
        else (rows_arr, dst2, zeros)

    scratch = ([pltpu.VMEM((k,), jnp.int32)] * (2 * nbuf)
               + [pltpu.VMEM((k, D), jnp.float32)] * nbuf
               + [pltpu.VMEM_SHARED((npad, D), jnp.float32)]
               + [pltpu.SemaphoreType.DMA((nbuf,))] * 2)

    @functools.partial(
        pl.kernel,
        out_type=(jax.ShapeDtypeStruct((npad, D), jnp.float32),
                  jax.ShapeDtypeStruct((npad, D), jnp.float32)),
        mesh=_sc_mesh(),
        scratch_types=scratch,
    )
    def body(*refs):
        if gather:
            (tab_h, src_h, dst_h, z_h, o0_h, o1_h) = refs[:6]
            sc = refs[6:]
        else:
            (rows_h, dst_h, z_h, o0_h, o1_h) = refs[:5]
            sc = refs[5:]
        isrc = sc[0:nbuf]
        idst = sc[nbuf:2 * nbuf]
        rows = sc[2 * nbuf:3 * nbuf]
        acc = sc[3 * nbuf]
        isem, gsem = sc[3 * nbuf + 1:3 * nbuf + 3]
        c = lax.axis_index("c")
        s = lax.axis_index("s")
        w = c * NS + s
        base = w * ew

        pltpu.sync_copy(z_h, acc.at[pl.ds(s * rpt, rpt)])

        def idx_load(i, h):
            if gather:
                pltpu.async_copy(src_h.at[w * nch + i], isrc[h], isem.at[h])
            pltpu.async_copy(dst_h.at[w * nch + i], idst[h], isem.at[h])

        def idx_wait(h):
            if gather:
                pltpu.make_async_copy(src_h.at[0], isrc[h], isem.at[h]).wait()
            pltpu.make_async_copy(dst_h.at[0], idst[h], isem.at[h]).wait()

        def g_fire(i, h):
            if gather:
                pltpu.async_copy(tab_h.at[isrc[h]], rows[h], gsem.at[h])
            else:
                pltpu.async_copy(rows_h.at[pl.ds(base + i * k, k)],
                                 rows[h], gsem.at[h])

        def g_wait(h):
            if gather:
                pltpu.make_async_copy(tab_h.at[isrc[h]], rows[h],
                                      gsem.at[h]).wait()
            else:
                pltpu.make_async_copy(rows_h.at[pl.ds(0, k)], rows[h],
                                      gsem.at[h]).wait()

        def chunk_body(x, sh, traced):
            nxt = x + nbuf - 1
            so = (sh - 1) % nbuf

            def fire_next():
                idx_wait(so)
                g_fire(nxt, so)

            if traced:
                pl.when(nxt < nch)(fire_next)
            elif nxt < nch:
                fire_next()
            g_wait(sh)
            pltpu.sync_copy(rows[sh], acc.at[idst[sh]], add=True)

            def load_next():
                idx_load(x + nbuf, sh)
            if traced:
                pl.when(x + nbuf < nch)(load_next)
            elif x + nbuf < nch:
                load_next()

        for j in range(nbuf):
            idx_load(j, j)
        plsc.subcore_barrier()
        for j in range(nbuf - 1):
            idx_wait(j)
            g_fire(j, j)

        def block(g, _):
            xb = g * nbuf
            for sh in range(nbuf):
                chunk_body(xb + sh, sh, True)
            return 0

        nmain = (nch // nbuf) * nbuf
        lax.fori_loop(0, nch // nbuf, block, 0)
        for j in range(nch % nbuf):
            chunk_body(nmain + j, (nmain + j) % nbuf, False)
        plsc.subcore_barrier()

        @pl.when(c == 0)
        def _():
            pltpu.sync_copy(acc.at[pl.ds(s * rpt, rpt)],
                            o0_h.at[pl.ds(s * rpt, rpt)])

        @pl.when(c == 1)
        def _():
            pltpu.sync_copy(acc.at[pl.ds(s * rpt, rpt)],
                            o1_h.at[pl.ds(s * rpt, rpt)])

    return body(*ins)


def _sc_gather_segsum(table, src, dst, nseg):
    return _sc_segsum(None, dst, nseg, table=table, src=src)


def _attention(agt, agt_cp, ctx, ctx_cp, src, dst, Wd128, Wc_top, Wc_bot, Wq, Wo):
    nseg = agt.shape[0]
    src_p = _pad_edges(src, 0)
    dst_g = _pad_edges(dst, 0)
    dst_s = _pad_edges(dst, nseg)
    gcs_t = _tc_proj_cs(ctx, ctx_cp, Wc_top, Wd128)
    ad_t = _tc_matmul(agt_cp, Wd128)
    GCS, AD = _sc_gather2(gcs_t, ad_t, src_p, dst_g)
    msg = _tc_msg(GCS, AD, Wc_bot)
    p0, p1 = _sc_segsum(msg, dst_s, nseg)
    return _tc_att_update(agt, Wq, Wo, p0, p1)


def _pad_ctrs(ctrs):
    n = ctrs.shape[0]
    return jnp.concatenate(
        [ctrs, jnp.zeros((n, D - ctrs.shape[1]), ctrs.dtype)], axis=1)


def _pad_wd(Wd):
    return jnp.concatenate(
        [Wd, jnp.zeros((D - Wd.shape[0], Wd.shape[1]), Wd.dtype)], axis=0)


def kernel(actor_feats, actor_ctrs, node_feats, node_ctrs,
           W_actor, W_map1, W_map2,
           a2m_Wd, a2m_Wc, a2m_Wq, a2m_Wo,
           m2a_Wd, m2a_Wc, m2a_Wq, m2a_Wo,
           a2a_Wd, a2a_Wc, a2a_Wq, a2a_Wo,
           W_pred, rot, orig,
           map_src, map_dst, a2m_src, a2m_dst,
           m2a_src, m2a_dst, a2a_src, a2a_dst):
    i32 = jnp.int32
    map_src, map_dst = map_src.astype(i32), map_dst.astype(i32)
    a2m_src, a2m_dst = a2m_src.astype(i32), a2m_dst.astype(i32)
    m2a_src, m2a_dst = m2a_src.astype(i32), m2a_dst.astype(i32)
    a2a_src, a2a_dst = a2a_src.astype(i32), a2a_dst.astype(i32)

    actor_cp = _pad_ctrs(actor_ctrs)
    node_cp = _pad_ctrs(node_ctrs)
    map_src_p = _pad_edges(map_src, 0)
    map_dst_p = _pad_edges(map_dst, N_M)

    actors = _tc_encode(actor_feats, W_actor)

    nodes = node_feats
    for _ in range(2):
        P = _tc_matmul(nodes, W_map2)
        p0, p1 = _sc_gather_segsum(P, map_src_p, map_dst_p, N_M)
        nodes = _tc_map_update(nodes, W_map1, p0, p1)

    nodes = _attention(nodes, node_cp, actors, actor_cp, a2m_src, a2m_dst,
                       _pad_wd(a2m_Wd), a2m_Wc[:D], a2m_Wc[D:], a2m_Wq, a2m_Wo)

    P = _tc_matmul(nodes, W_map2)
    p0, p1 = _sc_gather_segsum(P, map_src_p, map_dst_p, N_M)
    nodes = _tc_map_update(nodes, W_map1, p0, p1)

    actors = _attention(actors, actor_cp, nodes, node_cp, m2a_src, m2a_dst,
                        _pad_wd(m2a_Wd), m2a_Wc[:D], m2a_Wc[D:], m2a_Wq, m2a_Wo)

    actors = _attention(actors, actor_cp, actors, actor_cp, a2a_src, a2a_dst,
                        _pad_wd(a2a_Wd), a2a_Wc[:D], a2a_Wc[D:], a2a_Wq, a2a_Wo)

    f = NUM_MODS * NUM_PREDS * 2
    fp = 384
    W2 = (W_pred.reshape(D, NUM_MODS, NUM_PREDS, 2) @ rot).reshape(D, f)
    W2 = jnp.concatenate([W2, jnp.zeros((D, fp - f), W2.dtype)], axis=1)
    bvec = jnp.tile(orig, f // 2)
    bvec = jnp.concatenate([bvec, jnp.zeros((fp - f,), bvec.dtype)])[None, :]
    reg = _tc_pred(actors, W2, bvec)
    return reg[:, :f].reshape(N_A, NUM_MODS, NUM_PREDS, 2)


# speedup vs baseline: 1.8939x; 1.8939x over previous
import functools

import jax
import jax.numpy as jnp
from jax import lax
from jax.experimental import pallas as pl
from jax.experimental.pallas import tpu as pltpu
from jax.experimental.pallas import tpu_sc as plsc

D = 128
NC, NS = 2, 16
NW = NC * NS
N_A, N_M = 1000, 10000
NUM_MODS, NUM_PREDS = 6, 30


SC_K = 128


def _pad_edges(idx, fill):
    e = idx.shape[0]
    g = NW * SC_K * 2
    ep = -(-e // g) * g
    if ep != e:
        idx = jnp.concatenate([idx, jnp.full((ep - e,), fill, idx.dtype)])
    return idx


def _seg_pad(nseg):
    rpt = -(-nseg // NS)
    rpt = (rpt + 7) // 8 * 8
    return rpt, rpt * NS


def _blk(m):
    for b in (512, 256, 200, 128, 8):
        if m % b == 0:
            return b
    raise ValueError(m)


def _enc_body(x, w, o):
    o[...] = jax.nn.relu(x[...] @ w[...])


def _tc_encode(x, W):
    m = x.shape[0]
    b = _blk(m)
    return pl.pallas_call(
        _enc_body,
        grid=(m // b,),
        in_specs=[pl.BlockSpec((b, D), lambda i: (i, 0)),
                  pl.BlockSpec((D, D), lambda i: (0, 0))],
        out_specs=pl.BlockSpec((b, D), lambda i: (i, 0)),
        out_shape=jax.ShapeDtypeStruct((m, D), jnp.float32),
    )(x, W)


def _mm_body(x, w, o):
    o[...] = x[...] @ w[...]


def _tc_matmul(x, W):
    m = x.shape[0]
    b = _blk(m)
    return pl.pallas_call(
        _mm_body,
        grid=(m // b,),
        in_specs=[pl.BlockSpec((b, D), lambda i: (i, 0)),
                  pl.BlockSpec((D, D), lambda i: (0, 0))],
        out_specs=pl.BlockSpec((b, D), lambda i: (i, 0)),
        out_shape=jax.ShapeDtypeStruct((m, D), jnp.float32),
    )(x, W)


def _map_upd_body(x, w1, a0, a1, o):
    o[...] = jax.nn.relu(x[...] @ w1[...] + (a0[...] + a1[...])) + x[...]


def _tc_map_update(x, W1, p0, p1):
    m = x.shape[0]
    b = _blk(m)
    return pl.pallas_call(
        _map_upd_body,
        grid=(m // b,),
        in_specs=[pl.BlockSpec((b, D), lambda i: (i, 0)),
                  pl.BlockSpec((D, D), lambda i: (0, 0)),
                  pl.BlockSpec((b, D), lambda i: (i, 0)),
                  pl.BlockSpec((b, D), lambda i: (i, 0))],
        out_specs=pl.BlockSpec((b, D), lambda i: (i, 0)),
        out_shape=jax.ShapeDtypeStruct((m, D), jnp.float32),
    )(x, W1, p0, p1)


def _att_upd_body(x, wq, wo, a0, a1, o):
    o[...] = jax.nn.relu(x[...] @ wq[...] + (a0[...] + a1[...]) @ wo[...]) + x[...]


def _tc_att_update(x, Wq, Wo, p0, p1):
    m = x.shape[0]
    b = _blk(m)
    return pl.pallas_call(
        _att_upd_body,
        grid=(m // b,),
        in_specs=[pl.BlockSpec((b, D), lambda i: (i, 0)),
                  pl.BlockSpec((D, D), lambda i: (0, 0)),
                  pl.BlockSpec((D, D), lambda i: (0, 0)),
                  pl.BlockSpec((b, D), lambda i: (i, 0)),
                  pl.BlockSpec((b, D), lambda i: (i, 0))],
        out_specs=pl.BlockSpec((b, D), lambda i: (i, 0)),
        out_shape=jax.ShapeDtypeStruct((m, D), jnp.float32),
    )(x, Wq, Wo, p0, p1)


def _proj_cs_body(x, cp, wt, wd, o):
    o[:, :D] = x[...] @ wt[...]
    o[:, D:] = cp[...] @ wd[...]


def _tc_proj_cs(x, cp, Wt, Wd):
    m = x.shape[0]
    b = _blk(m)
    return pl.pallas_call(
        _proj_cs_body,
        grid=(m // b,),
        in_specs=[pl.BlockSpec((b, D), lambda i: (i, 0)),
                  pl.BlockSpec((b, D), lambda i: (i, 0)),
                  pl.BlockSpec((D, D), lambda i: (0, 0)),
                  pl.BlockSpec((D, D), lambda i: (0, 0))],
        out_specs=pl.BlockSpec((b, 2 * D), lambda i: (i, 0)),
        out_shape=jax.ShapeDtypeStruct((m, 2 * D), jnp.float32),
    )(x, cp, Wt, Wd)


def _msg_body(gcs, ad, wcb, o):
    df = jax.nn.relu(ad[...] - gcs[:, D:])
    o[...] = jax.nn.relu(gcs[:, :D] + df @ wcb[...])


def _tc_msg(GCS, AD, Wcb):
    e = GCS.shape[0]
    b = 256
    return pl.pallas_call(
        _msg_body,
        grid=(e // b,),
        in_specs=[pl.BlockSpec((b, 2 * D), lambda i: (i, 0)),
                  pl.BlockSpec((b, D), lambda i: (i, 0)),
                  pl.BlockSpec((D, D), lambda i: (0, 0))],
        out_specs=pl.BlockSpec((b, D), lambda i: (i, 0)),
        out_shape=jax.ShapeDtypeStruct((e, D), jnp.float32),
    )(GCS, AD, Wcb)


def _pred_body(x, w, bvec, o):
    o[...] = x[...] @ w[...] + bvec[...]


def _tc_pred(x, W, bvec):
    m, f = x.shape[0], W.shape[1]
    b = _blk(m)
    return pl.pallas_call(
        _pred_body,
        grid=(m // b,),
        in_specs=[pl.BlockSpec((b, D), lambda i: (i, 0)),
                  pl.BlockSpec((D, f), lambda i: (0, 0)),
                  pl.BlockSpec((1, f), lambda i: (0, 0))],
        out_specs=pl.BlockSpec((b, f), lambda i: (i, 0)),
        out_shape=jax.ShapeDtypeStruct((m, f), jnp.float32),
    )(x, W, bvec)


@functools.lru_cache(maxsize=1)
def _sc_mesh():
    return plsc.VectorSubcoreMesh(core_axis_name="c", subcore_axis_name="s",
                                  num_cores=NC, num_subcores=NS)


SC_NBUF = 2
SC_CK = 128
SC0_FRAC = 2.0 / 3.0


def _core_split(e, frac=SC0_FRAC):
    p = e // SC_CK // NW
    nc0 = int(round(p * frac / 2)) * 2
    nc0 = max(SC_NBUF, min(nc0, p - SC_NBUF))
    nc1 = p - nc0
    assert nc0 % 2 == 0 and nc1 % 2 == 0 and nc0 >= 2 and nc1 >= 2
    return nc0, nc1


def _sc_gather2(gcs_t, ad_t, src, dst):
    e = src.shape[0]
    k = SC_CK
    nbuf = SC_NBUF
    nch = e // k
    assert nch * k == e
    nc0, nc1 = _core_split(e)
    src2 = src.reshape(nch, k)
    dst2 = dst.reshape(nch, k)

    scratch = ([pltpu.VMEM((k,), jnp.int32)] * (2 * nbuf)
               + [pltpu.VMEM((k, 2 * D), jnp.float32)] * nbuf
               + [pltpu.VMEM((k, D), jnp.float32)] * nbuf
               + [pltpu.SemaphoreType.DMA((nbuf,))] * 3)

    @functools.partial(
        pl.kernel,
        out_type=(jax.ShapeDtypeStruct((e, 2 * D), jnp.float32),
                  jax.ShapeDtypeStruct((e, D), jnp.float32)),
        mesh=_sc_mesh(),
        scratch_types=scratch,
    )
    def body(gcst_h, adt_h, src_h, dst_h, gcs_h, ad_h, *sc):
        isrc = sc[0:nbuf]
        idst = sc[nbuf:2 * nbuf]
        gb = sc[2 * nbuf:3 * nbuf]
        ab = sc[3 * nbuf:4 * nbuf]
        isem, gsem, osem = sc[4 * nbuf:4 * nbuf + 3]
        c = lax.axis_index("c")
        s = lax.axis_index("s")
        nchw = jnp.where(c == 0, nc0, nc1)
        cb = jnp.where(c == 0, s * nc0, NS * nc0 + s * nc1)

        def idx_load(i, h):
            pltpu.async_copy(src_h.at[cb + i], isrc[h], isem.at[h])
            pltpu.async_copy(dst_h.at[cb + i], idst[h], isem.at[h])

        def idx_wait(h):
            pltpu.make_async_copy(src_h.at[0], isrc[h], isem.at[h]).wait()
            pltpu.make_async_copy(dst_h.at[0], idst[h], isem.at[h]).wait()

        def g_fire(h):
            pltpu.async_copy(gcst_h.at[isrc[h]], gb[h], gsem.at[h])
            pltpu.async_copy(adt_h.at[idst[h]], ab[h], gsem.at[h])

        def g_wait(h):
            pltpu.make_async_copy(gcst_h.at[isrc[h]], gb[h], gsem.at[h]).wait()
            pltpu.make_async_copy(adt_h.at[idst[h]], ab[h], gsem.at[h]).wait()

        def out_fire(i, h):
            off = (cb + i) * k
            pltpu.async_copy(gb[h], gcs_h.at[pl.ds(off, k)], osem.at[h])
            pltpu.async_copy(ab[h], ad_h.at[pl.ds(off, k)], osem.at[h])

        def out_wait(h):
            pltpu.make_async_copy(gb[h], gcs_h.at[pl.ds(0, k)],
                                  osem.at[h]).wait()
            pltpu.make_async_copy(ab[h], ad_h.at[pl.ds(0, k)],
                                  osem.at[h]).wait()

        def chunk_body(x, sh):
            nxt = x + nbuf - 1
            so = (sh - 1) % nbuf

            def fire_next():
                @pl.when(nxt >= nbuf)
                def _():
                    out_wait(so)
                idx_wait(so)
                g_fire(so)

            pl.when(nxt < nchw)(fire_next)
            g_wait(sh)
            out_fire(x, sh)

            def load_next():
                idx_load(x + nbuf, sh)
            pl.when(x + nbuf < nchw)(load_next)

        for j in range(nbuf):
            idx_load(j, j)
        for j in range(nbuf - 1):
            idx_wait(j)
            g_fire(j)

        def block(g, _):
            xb = g * nbuf
            for sh in range(nbuf):
                chunk_body(xb + sh, sh)
            return 0

        lax.fori_loop(0, nchw // nbuf, block, 0)
        for j in range(nbuf):
            pltpu.make_async_copy(gb[j], gcs_h.at[pl.ds(0, k)],
                                  osem.at[j]).wait()
            pltpu.make_async_copy(ab[j], ad_h.at[pl.ds(0, k)],
                                  osem.at[j]).wait()

    return body(gcs_t, ad_t, src2, dst2)


def _sc_segsum(rows_arr, dst, nseg, table=None, src=None):
    e = dst.shape[0]
    k = SC_CK
    nbuf = SC_NBUF
    nch = e // k
    assert nch * k == e
    gather = table is not None
    nc0, nc1 = _core_split(e, SC0_FRAC if gather else 0.5)
    rpt, npad = _seg_pad(nseg)
    zeros = jnp.zeros((rpt, D), jnp.float32)
    dst2 = dst.reshape(nch, k)
    ins = (table, src.reshape(nch, k), dst2, zeros) if gather \
        else (rows_arr, dst2, zeros)

    scratch = ([pltpu.VMEM((k,), jnp.int32)] * (2 * nbuf)
               + [pltpu.VMEM((k, D), jnp.float32)] * nbuf
               + [pltpu.VMEM_SHARED((npad, D), jnp.float32)]
               + [pltpu.SemaphoreType.DMA((nbuf,))] * 2)

    @functools.partial(
        pl.kernel,
        out_type=(jax.ShapeDtypeStruct((npad, D), jnp.float32),
                  jax.ShapeDtypeStruct((npad, D), jnp.float32)),
        mesh=_sc_mesh(),
        scratch_types=scratch,
    )
    def body(*refs):
        if gather:
            (tab_h, src_h, dst_h, z_h, o0_h, o1_h) = refs[:6]
            sc = refs[6:]
        else:
            (rows_h, dst_h, z_h, o0_h, o1_h) = refs[:5]
            sc = refs[5:]
        isrc = sc[0:nbuf]
        idst = sc[nbuf:2 * nbuf]
        rows = sc[2 * nbuf:3 * nbuf]
        acc = sc[3 * nbuf]
        isem, gsem = sc[3 * nbuf + 1:3 * nbuf + 3]
        c = lax.axis_index("c")
        s = lax.axis_index("s")
        nchw = jnp.where(c == 0, nc0, nc1)
        cb = jnp.where(c == 0, s * nc0, NS * nc0 + s * nc1)

        pltpu.sync_copy(z_h, acc.at[pl.ds(s * rpt, rpt)])

        def idx_load(i, h):
            if gather:
                pltpu.async_copy(src_h.at[cb + i], isrc[h], isem.at[h])
            pltpu.async_copy(dst_h.at[cb + i], idst[h], isem.at[h])

        def idx_wait(h):
            if gather:
                pltpu.make_async_copy(src_h.at[0], isrc[h], isem.at[h]).wait()
            pltpu.make_async_copy(dst_h.at[0], idst[h], isem.at[h]).wait()

        def g_fire(i, h):
            if gather:
                pltpu.async_copy(tab_h.at[isrc[h]], rows[h], gsem.at[h])
            else:
                pltpu.async_copy(rows_h.at[pl.ds((cb + i) * k, k)],
                                 rows[h], gsem.at[h])

        def g_wait(h):
            if gather:
                pltpu.make_async_copy(tab_h.at[isrc[h]], rows[h],
                                      gsem.at[h]).wait()
            else:
                pltpu.make_async_copy(rows_h.at[pl.ds(0, k)], rows[h],
                                      gsem.at[h]).wait()

        def chunk_body(x, sh):
            nxt = x + nbuf - 1
            so = (sh - 1) % nbuf

            def fire_next():
                idx_wait(so)
                g_fire(nxt, so)

            pl.when(nxt < nchw)(fire_next)
            g_wait(sh)
            pltpu.sync_copy(rows[sh], acc.at[idst[sh]], add=True)

            def load_next():
                idx_load(x + nbuf, sh)
            pl.when(x + nbuf < nchw)(load_next)

        for j in range(nbuf):
            idx_load(j, j)
        plsc.subcore_barrier()
        for j in range(nbuf - 1):
            idx_wait(j)
            g_fire(j, j)

        def block(g, _):
            xb = g * nbuf
            for sh in range(nbuf):
                chunk_body(xb + sh, sh)
            return 0

        lax.fori_loop(0, nchw // nbuf, block, 0)
        plsc.subcore_barrier()

        @pl.when(c == 0)
        def _():
            pltpu.sync_copy(acc.at[pl.ds(s * rpt, rpt)],
                            o0_h.at[pl.ds(s * rpt, rpt)])

        @pl.when(c == 1)
        def _():
            pltpu.sync_copy(acc.at[pl.ds(s * rpt, rpt)],
                            o1_h.at[pl.ds(s * rpt, rpt)])

    return body(*ins)


def _sc_gather_segsum(table, src, dst, nseg):
    return _sc_segsum(None, dst, nseg, table=table, src=src)


def _attention(agt, agt_cp, ctx, ctx_cp, src, dst, Wd128, Wc_top, Wc_bot, Wq, Wo):
    nseg = agt.shape[0]
    src_p = _pad_edges(src, 0)
    dst_g = _pad_edges(dst, 0)
    dst_s = _pad_edges(dst, nseg)
    gcs_t = _tc_proj_cs(ctx, ctx_cp, Wc_top, Wd128)
    ad_t = _tc_matmul(agt_cp, Wd128)
    GCS, AD = _sc_gather2(gcs_t, ad_t, src_p, dst_g)
    msg = _tc_msg(GCS, AD, Wc_bot)
    p0, p1 = _sc_segsum(msg, dst_s, nseg)
    return _tc_att_update(agt, Wq, Wo, p0, p1)


def _pad_ctrs(ctrs):
    n = ctrs.shape[0]
    return jnp.concatenate(
        [ctrs, jnp.zeros((n, D - ctrs.shape[1]), ctrs.dtype)], axis=1)


def _pad_wd(Wd):
    return jnp.concatenate(
        [Wd, jnp.zeros((D - Wd.shape[0], Wd.shape[1]), Wd.dtype)], axis=0)


def kernel(actor_feats, actor_ctrs, node_feats, node_ctrs,
           W_actor, W_map1, W_map2,
           a2m_Wd, a2m_Wc, a2m_Wq, a2m_Wo,
           m2a_Wd, m2a_Wc, m2a_Wq, m2a_Wo,
           a2a_Wd, a2a_Wc, a2a_Wq, a2a_Wo,
           W_pred, rot, orig,
           map_src, map_dst, a2m_src, a2m_dst,
           m2a_src, m2a_dst, a2a_src, a2a_dst):
    i32 = jnp.int32
    map_src, map_dst = map_src.astype(i32), map_dst.astype(i32)
    a2m_src, a2m_dst = a2m_src.astype(i32), a2m_dst.astype(i32)
    m2a_src, m2a_dst = m2a_src.astype(i32), m2a_dst.astype(i32)
    a2a_src, a2a_dst = a2a_src.astype(i32), a2a_dst.astype(i32)

    actor_cp = _pad_ctrs(actor_ctrs)
    node_cp = _pad_ctrs(node_ctrs)
    map_src_p = _pad_edges(map_src, 0)
    map_dst_p = _pad_edges(map_dst, N_M)

    actors = _tc_encode(actor_feats, W_actor)

    nodes = node_feats
    for _ in range(2):
        P = _tc_matmul(nodes, W_map2)
        p0, p1 = _sc_gather_segsum(P, map_src_p, map_dst_p, N_M)
        nodes = _tc_map_update(nodes, W_map1, p0, p1)

    nodes = _attention(nodes, node_cp, actors, actor_cp, a2m_src, a2m_dst,
                       _pad_wd(a2m_Wd), a2m_Wc[:D], a2m_Wc[D:], a2m_Wq, a2m_Wo)

    P = _tc_matmul(nodes, W_map2)
    p0, p1 = _sc_gather_segsum(P, map_src_p, map_dst_p, N_M)
    nodes = _tc_map_update(nodes, W_map1, p0, p1)

    actors = _attention(actors, actor_cp, nodes, node_cp, m2a_src, m2a_dst,
                        _pad_wd(m2a_Wd), m2a_Wc[:D], m2a_Wc[D:], m2a_Wq, m2a_Wo)

    actors = _attention(actors, actor_cp, actors, actor_cp, a2a_src, a2a_dst,
                        _pad_wd(a2a_Wd), a2a_Wc[:D], a2a_Wc[D:], a2a_Wq, a2a_Wo)

    f = NUM_MODS * NUM_PREDS * 2
    fp = 384
    W2 = (W_pred.reshape(D, NUM_MODS, NUM_PREDS, 2) @ rot).reshape(D, f)
    W2 = jnp.concatenate([W2, jnp.zeros((D, fp - f), W2.dtype)], axis=1)
    bvec = jnp.tile(orig, f // 2)
    bvec = jnp.concatenate([bvec, jnp.zeros((fp - f,), bvec.dtype)])[None, :]
    reg = _tc_pred(actors, W2, bvec)
    return reg[:, :f].reshape(N_A, NUM_MODS, NUM_PREDS, 2)
